# baseline (speedup 1.0000x reference)
import functools

import jax
import jax.numpy as jnp
from jax import lax
from jax.experimental import pallas as pl
from jax.experimental.pallas import tpu as pltpu
from jax.experimental.pallas import tpu_sc as plsc

_INFO = plsc.get_sparse_core_info()
_NC = _INFO.num_cores
_NS = _INFO.num_subcores
_NW = _NC * _NS

_TOK = 256


@functools.lru_cache(maxsize=None)
def _make_kernel(n_l: int, n_b: int, embed: int):
    assert embed == 64 and n_b % _TOK == 0
    n_j2 = n_b // _TOK
    units = n_l * n_j2
    assert units % _NW == 0
    per_w = units // _NW
    assert per_w % 2 == 0 and per_w >= 6

    mesh = plsc.VectorSubcoreMesh(core_axis_name="c", subcore_axis_name="s")

    @functools.partial(
        pl.kernel,
        mesh=mesh,
        out_type=jax.ShapeDtypeStruct((n_l, 8, n_b // 128, 8, 128),
                                      jnp.float32),
        scratch_types=[
            pltpu.VMEM((per_w * _TOK,), jnp.int32),
            pltpu.VMEM((_TOK, embed), jnp.float32),
            pltpu.VMEM((_TOK, embed), jnp.float32),
            pltpu.VMEM((8, 2, 8, 128), jnp.float32),
            pltpu.VMEM((8, 2, 8, 128), jnp.float32),
            pltpu.SemaphoreType.DMA,
            pltpu.SemaphoreType.DMA,
            pltpu.SemaphoreType.DMA,
            pltpu.SemaphoreType.DMA,
        ],
        compiler_params=pltpu.CompilerParams(use_tc_tiling_on_sc=False,
                                             needs_layout_passes=False),
    )
    def gather_kernel(idx_hbm, table_hbm, out_hbm, idx_v, rows_a, rows_b,
                      t_a, t_b, sem_ga, sem_gb, sem_sa, sem_sb):
        wid = lax.axis_index("s") * _NC + lax.axis_index("c")
        u0 = wid * per_w
        pltpu.sync_copy(idx_hbm.at[pl.ds(u0 * _TOK, per_w * _TOK)], idx_v)

        rows = (rows_a, rows_b)
        tbuf = (t_a, t_b)
        sem_g = (sem_ga, sem_gb)
        sem_s = (sem_sa, sem_sb)

        row_idx = tuple(lax.iota(jnp.int32, 16) + k * 16 for k in range(16))

        def start_gather(i, s):
            return pltpu.async_copy(
                table_hbm.at[idx_v.at[pl.ds(i * _TOK, _TOK)]],
                rows[s], sem_g[s])

        def start_store(i, s):
            u = u0 + i
            l = u // n_j2
            j2 = u - l * n_j2
            return pltpu.async_copy(
                tbuf[s], out_hbm.at[l, :, pl.ds(j2 * 2, 2)], sem_s[s])

        def wait_gather(s):
            pltpu.make_async_copy(table_hbm.at[idx_v.at[pl.ds(0, _TOK)]],
                                  rows[s], sem_g[s]).wait()

        def wait_store(s):
            pltpu.make_async_copy(tbuf[s], out_hbm.at[0, :, pl.ds(0, 2)],
                                  sem_s[s]).wait()

        def transpose_unit(s):
            rows_ref = rows[s]
            t_ref = tbuf[s]

            def body_d(d, carry):
                dvec = jnp.full((16,), d, jnp.int32)
                di = d // 8
                dr = d - di * 8
                for k in range(16):
                    val = plsc.load_gather(rows_ref, [row_idx[k], dvec])
                    t_ref[di, k // 8, dr, pl.ds((k % 8) * 16, 16)] = val
                return carry

            lax.fori_loop(0, embed, body_d, 0, unroll=False)

        start_gather(0, 0)
        start_gather(1, 1)
        for i in (0, 1):
            s = i % 2
            wait_gather(s)
            transpose_unit(s)
            start_store(i, s)
            start_gather(i + 2, s)

        def body(jj, carry):
            for p in range(2):
                i = 2 + 2 * jj + p
                s = p
                wait_gather(s)
                wait_store(s)
                transpose_unit(s)
                start_store(i, s)
                start_gather(i + 2, s)
            return carry

        lax.fori_loop(0, (per_w - 4) // 2, body, 0, unroll=False)

        for i in (per_w - 2, per_w - 1):
            s = i % 2
            wait_gather(s)
            wait_store(s)
            transpose_unit(s)
            start_store(i, s)
        wait_store(0)
        wait_store(1)

    return gather_kernel


def kernel(token_id, table):
    b, s = token_id.shape
    v, d = table.shape
    flat_t = jnp.transpose(token_id).reshape(-1).astype(jnp.int32)
    out5 = _make_kernel(s, b, d)(flat_t, table)
    return out5.transpose(2, 4, 0, 1, 3).reshape(b, s, d)


# speedup vs baseline: 1.1452x; 1.1452x over previous
import functools

import jax
import jax.numpy as jnp
from jax import lax
from jax.experimental import pallas as pl
from jax.experimental.pallas import tpu as pltpu
from jax.experimental.pallas import tpu_sc as plsc

_INFO = plsc.get_sparse_core_info()
_NC = _INFO.num_cores
_NS = _INFO.num_subcores
_NW = _NC * _NS

_TOK = 256


@functools.lru_cache(maxsize=None)
def _make_kernel(n_l: int, n_b: int, embed: int):
    assert embed == 64 and n_b % _TOK == 0
    n_j2 = n_b // _TOK
    units = n_l * n_j2
    assert units % _NW == 0
    per_w = units // _NW
    assert per_w % 2 == 0 and per_w >= 6
    t_words = 16 * 8 * 128

    mesh = plsc.VectorSubcoreMesh(core_axis_name="c", subcore_axis_name="s")

    @functools.partial(
        pl.kernel,
        mesh=mesh,
        out_type=jax.ShapeDtypeStruct((n_l * 8 * (n_b // 128) * 8 * 128,),
                                      jnp.float32),
        scratch_types=[
            pltpu.VMEM((per_w * _TOK,), jnp.int32),
            pltpu.VMEM((_TOK, embed), jnp.float32),
            pltpu.VMEM((_TOK, embed), jnp.float32),
            pltpu.VMEM((t_words,), jnp.float32),
            pltpu.VMEM((t_words,), jnp.float32),
            pltpu.SemaphoreType.DMA,
            pltpu.SemaphoreType.DMA,
            pltpu.SemaphoreType.DMA,
            pltpu.SemaphoreType.DMA,
        ],
        compiler_params=pltpu.CompilerParams(use_tc_tiling_on_sc=False,
                                             needs_layout_passes=False),
    )
    def gather_kernel(idx_hbm, table_hbm, out_hbm, idx_v, rows_a, rows_b,
                      t_a, t_b, sem_ga, sem_gb, sem_sa, sem_sb):
        wid = lax.axis_index("s") * _NC + lax.axis_index("c")
        u0 = wid * per_w
        pltpu.sync_copy(idx_hbm.at[pl.ds(u0 * _TOK, per_w * _TOK)], idx_v)

        rows = (rows_a, rows_b)
        tbuf = (t_a, t_b)
        sem_g = (sem_ga, sem_gb)
        sem_s = (sem_sa, sem_sb)

        iota = lax.iota(jnp.int32, 16)
        svec = tuple(
            (dg * 4096) + (iota // 8) * 2048 + (iota % 8) * 128
            for dg in range(4)
        )

        def start_gather(i, s):
            return pltpu.async_copy(
                table_hbm.at[idx_v.at[pl.ds(i * _TOK, _TOK)]],
                rows[s], sem_g[s])

        def start_store(i, s):
            u = u0 + i
            l = u // n_j2
            j2 = u - l * n_j2
            for blk in range(8):
                dst = ((l * 8 + blk) * (n_b // 128) + 2 * j2) * 1024
                pltpu.async_copy(tbuf[s].at[pl.ds(blk * 2048, 2048)],
                                 out_hbm.at[pl.ds(dst, 2048)], sem_s[s])

        def wait_gather(s):
            pltpu.make_async_copy(table_hbm.at[idx_v.at[pl.ds(0, _TOK)]],
                                  rows[s], sem_g[s]).wait()

        def wait_store(s):
            pltpu.make_async_copy(tbuf[s], out_hbm.at[pl.ds(0, t_words)],
                                  sem_s[s]).wait()

        def transpose_unit(s):
            rows_ref = rows[s]
            t_ref = tbuf[s]

            def body_n(n, carry):
                off = jnp.full((16,), carry, jnp.int32)
                for dg in range(4):
                    val = rows_ref[n, pl.ds(dg * 16, 16)]
                    plsc.store_scatter(t_ref, [svec[dg] + off], val)
                return carry + 1

            lax.fori_loop(0, 128, body_n, 0, unroll=4)
            lax.fori_loop(128, 256, body_n, 1024, unroll=4)

        start_gather(0, 0)
        start_gather(1, 1)
        for i in (0, 1):
            s = i % 2
            wait_gather(s)
            transpose_unit(s)
            start_store(i, s)
            start_gather(i + 2, s)

        def body(jj, carry):
            for p in range(2):
                i = 2 + 2 * jj + p
                s = p
                wait_gather(s)
                wait_store(s)
                transpose_unit(s)
                start_store(i, s)
                start_gather(i + 2, s)
            return carry

        lax.fori_loop(0, (per_w - 4) // 2, body, 0, unroll=False)

        for i in (per_w - 2, per_w - 1):
            s = i % 2
            wait_gather(s)
            wait_store(s)
            transpose_unit(s)
            start_store(i, s)
        wait_store(0)
        wait_store(1)

    return gather_kernel


def kernel(token_id, table):
    b, s = token_id.shape
    v, d = table.shape
    flat_t = jnp.transpose(token_id).reshape(-1).astype(jnp.int32)
    out1 = _make_kernel(s, b, d)(flat_t, table)
    out5 = out1.reshape(s, 8, b // 128, 8, 128)
    return out5.transpose(2, 4, 0, 1, 3).reshape(b, s, d)


# speedup vs baseline: 1.1691x; 1.0209x over previous
import functools

import jax
import jax.numpy as jnp
from jax import lax
from jax.experimental import pallas as pl
from jax.experimental.pallas import tpu as pltpu
from jax.experimental.pallas import tpu_sc as plsc

_INFO = plsc.get_sparse_core_info()
_NC = _INFO.num_cores
_NS = _INFO.num_subcores
_NW = _NC * _NS

_TOK = 256


@functools.lru_cache(maxsize=None)
def _make_kernel(n_l: int, n_b: int, embed: int):
    assert embed == 64 and n_b % _TOK == 0
    n_j2 = n_b // _TOK
    units = n_l * n_j2
    assert units % _NW == 0
    per_w = units // _NW
    assert per_w % 2 == 0 and per_w >= 6
    t_words = 16 * 8 * 128

    mesh = plsc.VectorSubcoreMesh(core_axis_name="c", subcore_axis_name="s")

    @functools.partial(
        pl.kernel,
        mesh=mesh,
        out_type=jax.ShapeDtypeStruct((n_l * 8 * (n_b // 128) * 8 * 128,),
                                      jnp.float32),
        scratch_types=[
            pltpu.VMEM((per_w * _TOK,), jnp.int32),
            pltpu.VMEM((_TOK, embed), jnp.float32),
            pltpu.VMEM((_TOK, embed), jnp.float32),
            pltpu.VMEM((t_words,), jnp.float32),
            pltpu.VMEM((t_words,), jnp.float32),
            pltpu.SemaphoreType.DMA,
            pltpu.SemaphoreType.DMA,
            pltpu.SemaphoreType.DMA,
            pltpu.SemaphoreType.DMA,
        ],
        compiler_params=pltpu.CompilerParams(use_tc_tiling_on_sc=False,
                                             needs_layout_passes=False),
    )
    def gather_kernel(idx_hbm, table_hbm, out_hbm, idx_v, rows_a, rows_b,
                      t_a, t_b, sem_ga, sem_gb, sem_sa, sem_sb):
        wid = lax.axis_index("s") * _NC + lax.axis_index("c")
        u0 = wid * per_w
        pltpu.sync_copy(idx_hbm.at[pl.ds(u0 * _TOK, per_w * _TOK)], idx_v)

        rows = (rows_a, rows_b)
        tbuf = (t_a, t_b)
        sem_g = (sem_ga, sem_gb)
        sem_s = (sem_sa, sem_sb)

        iota = lax.iota(jnp.int32, 16)
        svec = tuple(
            (dg * 4096) + (iota // 8) * 2048 + (iota % 8) * 128
            for dg in range(4)
        )

        def start_gather(i, s):
            return pltpu.async_copy(
                table_hbm.at[idx_v.at[pl.ds(i * _TOK, _TOK)]],
                rows[s], sem_g[s])

        def start_store(i, s):
            u = u0 + i
            l = u // n_j2
            j2 = u - l * n_j2
            for blk in range(8):
                dst = ((l * 8 + blk) * (n_b // 128) + 2 * j2) * 1024
                pltpu.async_copy(tbuf[s].at[pl.ds(blk * 2048, 2048)],
                                 out_hbm.at[pl.ds(dst, 2048)], sem_s[s])

        def wait_gather(s):
            pltpu.make_async_copy(table_hbm.at[idx_v.at[pl.ds(0, _TOK)]],
                                  rows[s], sem_g[s]).wait()

        def wait_store(s):
            pltpu.make_async_copy(tbuf[s], out_hbm.at[pl.ds(0, t_words)],
                                  sem_s[s]).wait()

        def transpose_unit(s):
            rows_ref = rows[s]
            t_ref = tbuf[s]

            def body_n(i, carry):
                n0 = 2 * i
                vals = []
                idxs = []
                for t in range(2):
                    off = jnp.full((16,), carry + t, jnp.int32)
                    for dg in range(4):
                        vals.append(rows_ref[n0 + t, pl.ds(dg * 16, 16)])
                        idxs.append(svec[dg] + off)
                for ix, val in zip(idxs, vals):
                    plsc.store_scatter(t_ref, [ix], val)
                return carry + 2

            lax.fori_loop(0, 64, body_n, 0, unroll=2)
            lax.fori_loop(64, 128, body_n, 1024, unroll=2)

        start_gather(0, 0)
        start_gather(1, 1)
        for i in (0, 1):
            s = i % 2
            wait_gather(s)
            transpose_unit(s)
            start_store(i, s)
            start_gather(i + 2, s)

        def body(jj, carry):
            for p in range(2):
                i = 2 + 2 * jj + p
                s = p
                wait_gather(s)
                wait_store(s)
                transpose_unit(s)
                start_store(i, s)
                start_gather(i + 2, s)
            return carry

        lax.fori_loop(0, (per_w - 4) // 2, body, 0, unroll=False)

        for i in (per_w - 2, per_w - 1):
            s = i % 2
            wait_gather(s)
            wait_store(s)
            transpose_unit(s)
            start_store(i, s)
        wait_store(0)
        wait_store(1)

    return gather_kernel


def kernel(token_id, table):
    b, s = token_id.shape
    v, d = table.shape
    flat_t = jnp.transpose(token_id).reshape(-1).astype(jnp.int32)
    out1 = _make_kernel(s, b, d)(flat_t, table)
    out5 = out1.reshape(s, 8, b // 128, 8, 128)
    return out5.transpose(2, 4, 0, 1, 3).reshape(b, s, d)


# speedup vs baseline: 1.6807x; 1.4376x over previous
import functools

import jax
import jax.numpy as jnp
from jax import lax
from jax.experimental import pallas as pl
from jax.experimental.pallas import tpu as pltpu
from jax.experimental.pallas import tpu_sc as plsc

_INFO = plsc.get_sparse_core_info()
_NC = _INFO.num_cores
_NS = _INFO.num_subcores
_NW = _NC * _NS

_TOK = 256


@functools.lru_cache(maxsize=None)
def _make_kernel(n_l: int, n_b: int, embed: int):
    assert embed == 64 and n_b % _TOK == 0
    n_j2 = n_b // _TOK
    units = n_l * n_j2
    assert units % _NW == 0
    per_w = units // _NW
    assert per_w % 2 == 0 and per_w >= 6
    t_words = 16 * 8 * 128

    mesh = plsc.VectorSubcoreMesh(core_axis_name="c", subcore_axis_name="s")

    @functools.partial(
        pl.kernel,
        mesh=mesh,
        out_type=jax.ShapeDtypeStruct((n_l * 8 * (n_b // 128) * 8 * 128,),
                                      jnp.float32),
        scratch_types=[
            pltpu.VMEM((per_w * _TOK,), jnp.int32),
            pltpu.VMEM((_TOK, embed), jnp.float32),
            pltpu.VMEM((_TOK, embed), jnp.float32),
            pltpu.VMEM((t_words,), jnp.float32),
            pltpu.VMEM((t_words,), jnp.float32),
            pltpu.SemaphoreType.DMA,
            pltpu.SemaphoreType.DMA,
            pltpu.SemaphoreType.DMA,
            pltpu.SemaphoreType.DMA,
        ],
        compiler_params=pltpu.CompilerParams(use_tc_tiling_on_sc=False,
                                             needs_layout_passes=False),
    )
    def gather_kernel(idx_hbm, table_hbm, out_hbm, idx_v, rows_a, rows_b,
                      t_a, t_b, sem_ga, sem_gb, sem_sa, sem_sb):
        wid = lax.axis_index("s") * _NC + lax.axis_index("c")
        u0 = wid * per_w
        pltpu.sync_copy(idx_hbm.at[pl.ds(u0 * _TOK, per_w * _TOK)], idx_v)

        rows = (rows_a, rows_b)
        tbuf = (t_a, t_b)
        sem_g = (sem_ga, sem_gb)
        sem_s = (sem_sa, sem_sb)

        iota = lax.iota(jnp.int32, 16)
        rot16 = tuple((iota + k) % 16 for k in range(16))
        col = tuple(iota + dg * 16 for dg in range(4))
        stv = tuple(
            (dg * 4096) + (iota // 8) * 2048 + (iota % 8) * 128
            for dg in range(4)
        )

        def start_gather(i, s):
            return pltpu.async_copy(
                table_hbm.at[idx_v.at[pl.ds(i * _TOK, _TOK)]],
                rows[s], sem_g[s])

        def start_store(i, s):
            u = u0 + i
            l = u // n_j2
            j2 = u - l * n_j2
            for blk in range(8):
                dst = ((l * 8 + blk) * (n_b // 128) + 2 * j2) * 1024
                pltpu.async_copy(tbuf[s].at[pl.ds(blk * 2048, 2048)],
                                 out_hbm.at[pl.ds(dst, 2048)], sem_s[s])

        def wait_gather(s):
            pltpu.make_async_copy(table_hbm.at[idx_v.at[pl.ds(0, _TOK)]],
                                  rows[s], sem_g[s]).wait()

        def wait_store(s):
            pltpu.make_async_copy(tbuf[s], out_hbm.at[pl.ds(0, t_words)],
                                  sem_s[s]).wait()

        def transpose_unit(s):
            rows_ref = rows[s]
            t_ref = tbuf[s]

            def body_g(g, carry):
                n0 = g * 16
                cbase = (g // 8) * 1024 + (g % 8) * 16
                for k in range(16):
                    rk = rot16[k] + n0
                    ck = rot16[k] + cbase
                    for dg in range(4):
                        val = plsc.load_gather(rows_ref, [rk, col[dg]])
                        plsc.store_scatter(t_ref, [stv[dg] + ck], val)
                return carry

            lax.fori_loop(0, 16, body_g, 0, unroll=False)

        start_gather(0, 0)
        start_gather(1, 1)
        for i in (0, 1):
            s = i % 2
            wait_gather(s)
            transpose_unit(s)
            start_store(i, s)
            start_gather(i + 2, s)

        def body(jj, carry):
            for p in range(2):
                i = 2 + 2 * jj + p
                s = p
                wait_gather(s)
                wait_store(s)
                transpose_unit(s)
                start_store(i, s)
                start_gather(i + 2, s)
            return carry

        lax.fori_loop(0, (per_w - 4) // 2, body, 0, unroll=False)

        for i in (per_w - 2, per_w - 1):
            s = i % 2
            wait_gather(s)
            wait_store(s)
            transpose_unit(s)
            start_store(i, s)
        wait_store(0)
        wait_store(1)

    return gather_kernel


def kernel(token_id, table):
    b, s = token_id.shape
    v, d = table.shape
    flat_t = jnp.transpose(token_id).reshape(-1).astype(jnp.int32)
    out1 = _make_kernel(s, b, d)(flat_t, table)
    out5 = out1.reshape(s, 8, b // 128, 8, 128)
    return out5.transpose(2, 4, 0, 1, 3).reshape(b, s, d)


# speedup vs baseline: 25.8922x; 15.4056x over previous
import functools

import jax
import jax.numpy as jnp
from jax import lax
from jax.experimental import pallas as pl
from jax.experimental.pallas import tpu as pltpu
from jax.experimental.pallas import tpu_sc as plsc

promise = lax.GatherScatterMode.PROMISE_IN_BOUNDS

_INFO = plsc.get_sparse_core_info()
_NC = _INFO.num_cores
_NS = _INFO.num_subcores
_NW = _NC * _NS

_TOK = 256


@functools.lru_cache(maxsize=None)
def _make_kernel(n_l: int, n_b: int, embed: int):
    assert embed == 64 and n_b % _TOK == 0
    n_j2 = n_b // _TOK
    units = n_l * n_j2
    assert units % _NW == 0
    per_w = units // _NW
    assert per_w % 2 == 0 and per_w >= 6
    t_words = 16 * 8 * 128

    mesh = plsc.VectorSubcoreMesh(core_axis_name="c", subcore_axis_name="s")

    @functools.partial(
        pl.kernel,
        mesh=mesh,
        out_type=jax.ShapeDtypeStruct((n_l * 8 * (n_b // 128) * 8 * 128,),
                                      jnp.float32),
        scratch_types=[
            pltpu.VMEM((per_w * _TOK,), jnp.int32),
            pltpu.VMEM((_TOK,), jnp.int32),
            pltpu.VMEM((_TOK,), jnp.int32),
            pltpu.VMEM((_TOK,), jnp.int32),
            pltpu.VMEM((_TOK,), jnp.int32),
            pltpu.VMEM((_TOK, 2 * embed), jnp.float32),
            pltpu.VMEM((_TOK, 2 * embed), jnp.float32),
            pltpu.VMEM((t_words,), jnp.float32),
            pltpu.VMEM((t_words,), jnp.float32),
            pltpu.SemaphoreType.DMA,
            pltpu.SemaphoreType.DMA,
            pltpu.SemaphoreType.DMA,
            pltpu.SemaphoreType.DMA,
        ],
        compiler_params=pltpu.CompilerParams(use_tc_tiling_on_sc=True,
                                             needs_layout_passes=False),
    )
    def gather_kernel(idx_hbm, table_hbm, out_hbm, idx_v, h_a, h_b,
                      pv_a, pv_b, rows_a, rows_b,
                      t_a, t_b, sem_ga, sem_gb, sem_sa, sem_sb):
        wid = lax.axis_index("s") * _NC + lax.axis_index("c")
        u0 = wid * per_w
        pltpu.sync_copy(idx_hbm.at[pl.ds(u0 * _TOK, per_w * _TOK)], idx_v)

        half = (h_a, h_b)
        pvb = (pv_a, pv_b)
        rows = (rows_a, rows_b)
        tbuf = (t_a, t_b)
        sem_g = (sem_ga, sem_gb)
        sem_s = (sem_sa, sem_sb)

        iota = lax.iota(jnp.int32, 16)
        rot16 = tuple((iota + k) % 16 for k in range(16))
        col = tuple(iota + dg * 16 for dg in range(4))
        stv = tuple(
            (dg * 4096) + (iota // 8) * 2048 + (iota % 8) * 128
            for dg in range(4)
        )

        def start_gather(i, s):
            def mk(k, carry):
                v = idx_v[pl.ds(i * _TOK + k * 16, 16)]
                half[s][pl.ds(k * 16, 16)] = lax.shift_right_logical(v, 1)
                pvb[s][pl.ds(k * 16, 16)] = (v & 1) * 64
                return carry

            lax.fori_loop(0, _TOK // 16, mk, 0, unroll=4)
            return pltpu.async_copy(
                table_hbm.at[half[s]], rows[s], sem_g[s])

        def start_store(i, s):
            u = u0 + i
            l = u // n_j2
            j2 = u - l * n_j2
            for blk in range(8):
                dst = ((l * 8 + blk) * (n_b // 128) + 2 * j2) * 1024
                pltpu.async_copy(tbuf[s].at[pl.ds(blk * 2048, 2048)],
                                 out_hbm.at[pl.ds(dst, 2048)], sem_s[s])

        def wait_gather(s):
            pltpu.make_async_copy(table_hbm.at[half[s]],
                                  rows[s], sem_g[s]).wait()

        def wait_store(s):
            pltpu.make_async_copy(tbuf[s], out_hbm.at[pl.ds(0, t_words)],
                                  sem_s[s]).wait()

        def transpose_unit(s):
            rows_ref = rows[s]
            t_ref = tbuf[s]
            pv_ref = pvb[s]

            def body_g(g, carry):
                n0 = g * 16
                cbase = (g // 8) * 1024 + (g % 8) * 16
                pg = pv_ref[pl.ds(n0, 16)]
                for k in range(16):
                    rk = rot16[k] + n0
                    ck = rot16[k] + cbase
                    pk = pg.at[rot16[k]].get(mode="promise_in_bounds")
                    for dg in range(4):
                        val = plsc.load_gather(rows_ref,
                                               [rk, col[dg] + pk])
                        plsc.store_scatter(t_ref, [stv[dg] + ck], val)
                return carry

            lax.fori_loop(0, 16, body_g, 0, unroll=False)

        start_gather(0, 0)
        start_gather(1, 1)
        for i in (0, 1):
            s = i % 2
            wait_gather(s)
            transpose_unit(s)
            start_store(i, s)
            start_gather(i + 2, s)

        def body(jj, carry):
            for p in range(2):
                i = 2 + 2 * jj + p
                s = p
                wait_gather(s)
                wait_store(s)
                transpose_unit(s)
                start_store(i, s)
                start_gather(i + 2, s)
            return carry

        lax.fori_loop(0, (per_w - 4) // 2, body, 0, unroll=False)

        for i in (per_w - 2, per_w - 1):
            s = i % 2
            wait_gather(s)
            wait_store(s)
            transpose_unit(s)
            start_store(i, s)
        wait_store(0)
        wait_store(1)

    return gather_kernel


def kernel(token_id, table):
    b, s = token_id.shape
    v, d = table.shape
    flat_t = jnp.transpose(token_id).reshape(-1).astype(jnp.int32)
    table2 = table.reshape(v // 2, 2 * d)
    out1 = _make_kernel(s, b, d)(flat_t, table2)
    out5 = out1.reshape(s, 8, b // 128, 8, 128)
    return out5.transpose(2, 4, 0, 1, 3).reshape(b, s, d)
